# baseline (speedup 1.0000x reference)
import math

import jax
import jax.numpy as jnp
from jax.experimental import pallas as pl
from jax.experimental.pallas import tpu as pltpu

B, T, DIM, E = 2, 2048, 1024, 64
THRESHOLD = 0.8
C = max(min(T, math.ceil(T * 1.25 / E)), 4)
TBLK = 256
NBLK = T // TBLK


EC = E * C


def _router_kernel(x_ref, w_ref, rsel_ref, cmod_ref, disp_ref, comb_ref,
                   aux_ref, carry_ref, accm_ref, accp_ref):
    b = pl.program_id(0)
    i = pl.program_id(1)

    @pl.when(i == 0)
    def _():
        carry_ref[...] = jnp.zeros_like(carry_ref)
        accm_ref[...] = jnp.zeros_like(accm_ref)
        accp_ref[...] = jnp.zeros_like(accp_ref)

    xb = x_ref[0]
    w = w_ref[...]
    gT = jax.lax.dot_general(w, xb, (((0,), (1,)), ((), ())),
                             preferred_element_type=jnp.float32)
    g = gT.T

    m = jnp.max(g, axis=-1, keepdims=True)
    ex = jnp.exp(g - m)
    tacc = ex
    width = E // 2
    while width >= 1:
        tacc = tacc[:, :width] + tacc[:, width:2 * width]
        width //= 2
    p = ex / tacc

    li = jax.lax.broadcasted_iota(jnp.int32, (1, E), 1)

    def lane_roll(a, sh):
        return pltpu.roll(a, sh % E, 1)

    a = p
    for k in (2, 4, 8, 16, 32, 64):
        j = k // 2
        while j >= 1:
            bitj0 = (li & j) == 0
            descd = (li & k) == 0
            part = jnp.where(bitj0, lane_roll(a, -j), lane_roll(a, j))
            mn = jnp.minimum(a, part)
            mx = jnp.maximum(a, part)
            a = jnp.where(bitj0 == descd, mx, mn)
            j //= 2
    psort = a

    csum = psort
    d = 1
    while d < E:
        csum = csum + jnp.where(li >= d, lane_roll(csum, d), 0.0)
        d *= 2
    csum = jnp.where(li == 2, lane_roll(csum, 1) + psort, csum)

    kc = jnp.maximum(
        jnp.sum((csum < THRESHOLD).astype(jnp.float32), axis=-1,
                keepdims=True), 1.0)
    kcl = jnp.broadcast_to(kc, (TBLK, E))
    tau = jnp.min(jnp.where(li.astype(jnp.float32) < kcl, psort, jnp.inf),
                  axis=-1, keepdims=True)
    gt = (p > tau).astype(jnp.float32)
    eq = (p == tau).astype(jnp.float32)
    need = kc - jnp.sum(gt, axis=-1, keepdims=True)
    pre = eq
    d = 1
    while d < E:
        pre = pre + jnp.where(li >= d, lane_roll(pre, d), 0.0)
        d *= 2
    pre = pre - eq
    keepf = gt + eq * (pre < need).astype(jnp.float32)
    renorm = jnp.maximum(jnp.sum(keepf * p, axis=-1, keepdims=True), 1e-9)
    wgt = (keepf * p) / renorm

    ii = jax.lax.broadcasted_iota(jnp.int32, (TBLK, TBLK), 0)
    jj = jax.lax.broadcasted_iota(jnp.int32, (TBLK, TBLK), 1)
    lstrict = (ii > jj).astype(jnp.float32)
    pos_local = jax.lax.dot_general(lstrict, keepf, (((1,), (0,)), ((), ())),
                                    preferred_element_type=jnp.float32)
    carry = carry_ref[0:1, :]
    pos = pos_local + carry
    colsum = jnp.sum(keepf, axis=0, keepdims=True)
    carry_ref[...] = jnp.broadcast_to(carry + colsum, carry_ref.shape)

    keep_cap = keepf * (pos < float(C)).astype(jnp.float32)
    pos_sel = pos * keep_cap + float(C) * (1.0 - keep_cap)

    rsel = rsel_ref[...]

    def expand(v):
        head = v.astype(jnp.bfloat16).astype(jnp.float32)
        tail = v - head
        return (jax.lax.dot_general(head, rsel, (((1,), (0,)), ((), ())),
                                    preferred_element_type=jnp.float32)
                + jax.lax.dot_general(tail, rsel, (((1,), (0,)), ((), ())),
                                      preferred_element_type=jnp.float32))

    pos_exp = jax.lax.dot_general(pos_sel, rsel, (((1,), (0,)), ((), ())),
                                  preferred_element_type=jnp.float32)
    wgt_exp = expand(wgt)
    cmod = cmod_ref[0:1, :]
    disp = (pos_exp == cmod).astype(jnp.float32)
    disp_ref[0] = disp
    comb_ref[0] = disp * wgt_exp

    accm_ref[...] = accm_ref[...] + jnp.broadcast_to(colsum, accm_ref.shape)
    accp_ref[...] = accp_ref[...] + jnp.broadcast_to(
        jnp.sum(p, axis=0, keepdims=True), accp_ref.shape)

    @pl.when(i == NBLK - 1)
    def _():
        term = jnp.sum(accm_ref[0:1, :] * accp_ref[0:1, :])
        prev = jnp.where(b == 0, 0.0, aux_ref[0, 0])
        aux_ref[...] = jnp.broadcast_to(
            prev + term * (E / (B * T * T)), aux_ref.shape)


def kernel(x, w_gating):
    lanes = jnp.arange(EC, dtype=jnp.int32)
    rsel = (lanes // C == jnp.arange(E, dtype=jnp.int32)[:, None]
            ).astype(jnp.float32)
    cmod = jnp.broadcast_to((lanes % C).astype(jnp.float32), (8, EC))
    disp, comb, aux = pl.pallas_call(
        _router_kernel,
        grid=(B, NBLK),
        in_specs=[pl.BlockSpec((1, TBLK, DIM), lambda b, i: (b, i, 0)),
                  pl.BlockSpec((DIM, E), lambda b, i: (0, 0)),
                  pl.BlockSpec((E, EC), lambda b, i: (0, 0)),
                  pl.BlockSpec((8, EC), lambda b, i: (0, 0))],
        out_specs=[pl.BlockSpec((1, TBLK, EC), lambda b, i: (b, i, 0)),
                   pl.BlockSpec((1, TBLK, EC), lambda b, i: (b, i, 0)),
                   pl.BlockSpec((8, 128), lambda b, i: (0, 0))],
        out_shape=[jax.ShapeDtypeStruct((B, T, EC), jnp.float32),
                   jax.ShapeDtypeStruct((B, T, EC), jnp.float32),
                   jax.ShapeDtypeStruct((8, 128), jnp.float32)],
        scratch_shapes=[pltpu.VMEM((8, E), jnp.float32),
                        pltpu.VMEM((8, E), jnp.float32),
                        pltpu.VMEM((8, E), jnp.float32)],
        compiler_params=pltpu.CompilerParams(
            dimension_semantics=("arbitrary", "arbitrary")),
    )(x, w_gating, rsel, cmod)
    return (disp.reshape(B, T, E, C), comb.reshape(B, T, E, C),
            aux[0, 0].reshape(()))


# speedup vs baseline: 1.2013x; 1.2013x over previous
import math

import jax
import jax.numpy as jnp
from jax.experimental import pallas as pl
from jax.experimental.pallas import tpu as pltpu

B, T, DIM, E = 2, 2048, 1024, 64
THRESHOLD = 0.8
C = max(min(T, math.ceil(T * 1.25 / E)), 4)
TBLK = 512
NBLK = T // TBLK


EC = E * C


def _router_kernel(x_ref, w_ref, rsel_ref, cmod_ref, disp_ref, comb_ref,
                   aux_ref, carry_ref, accm_ref, accp_ref):
    b = pl.program_id(0)
    i = pl.program_id(1)

    @pl.when(i == 0)
    def _():
        carry_ref[...] = jnp.zeros_like(carry_ref)
        accm_ref[...] = jnp.zeros_like(accm_ref)
        accp_ref[...] = jnp.zeros_like(accp_ref)

    xb = x_ref[0]
    w = w_ref[...]
    gT = jax.lax.dot_general(w, xb, (((0,), (1,)), ((), ())),
                             preferred_element_type=jnp.float32)
    g = gT.T

    m = jnp.max(g, axis=-1, keepdims=True)
    ex = jnp.exp(g - m)
    tacc = ex
    width = E // 2
    while width >= 1:
        tacc = tacc[:, :width] + tacc[:, width:2 * width]
        width //= 2
    p = ex / tacc

    li = jax.lax.broadcasted_iota(jnp.int32, (1, E), 1)

    def lane_roll(a, sh):
        return pltpu.roll(a, sh % E, 1)

    a = p
    for k in (2, 4, 8, 16, 32, 64):
        j = k // 2
        while j >= 1:
            bitj0 = (li & j) == 0
            descd = (li & k) == 0
            part = jnp.where(bitj0, lane_roll(a, -j), lane_roll(a, j))
            mn = jnp.minimum(a, part)
            mx = jnp.maximum(a, part)
            a = jnp.where(bitj0 == descd, mx, mn)
            j //= 2
    psort = a

    csum = psort
    d = 1
    while d < E:
        csum = csum + jnp.where(li >= d, lane_roll(csum, d), 0.0)
        d *= 2
    csum = jnp.where(li == 2, lane_roll(csum, 1) + psort, csum)

    kc = jnp.maximum(
        jnp.sum((csum < THRESHOLD).astype(jnp.float32), axis=-1,
                keepdims=True), 1.0)
    kcl = jnp.broadcast_to(kc, (TBLK, E))
    tau = jnp.min(jnp.where(li.astype(jnp.float32) < kcl, psort, jnp.inf),
                  axis=-1, keepdims=True)
    gt = (p > tau).astype(jnp.float32)
    eq = (p == tau).astype(jnp.float32)
    need = kc - jnp.sum(gt, axis=-1, keepdims=True)
    pre = eq
    d = 1
    while d < E:
        pre = pre + jnp.where(li >= d, lane_roll(pre, d), 0.0)
        d *= 2
    pre = pre - eq
    keepf = gt + eq * (pre < need).astype(jnp.float32)
    renorm = jnp.maximum(jnp.sum(keepf * p, axis=-1, keepdims=True), 1e-9)
    wgt = (keepf * p) / renorm

    ii = jax.lax.broadcasted_iota(jnp.int32, (TBLK, TBLK), 0)
    jj = jax.lax.broadcasted_iota(jnp.int32, (TBLK, TBLK), 1)
    lstrict = (ii > jj).astype(jnp.float32)
    pos_local = jax.lax.dot_general(lstrict, keepf, (((1,), (0,)), ((), ())),
                                    preferred_element_type=jnp.float32)
    carry = carry_ref[0:1, :]
    pos = pos_local + carry
    colsum = jnp.sum(keepf, axis=0, keepdims=True)
    carry_ref[...] = jnp.broadcast_to(carry + colsum, carry_ref.shape)

    keep_cap = keepf * (pos < float(C)).astype(jnp.float32)
    pos_sel = pos * keep_cap + float(C) * (1.0 - keep_cap)

    rsel = rsel_ref[...]

    def expand(v):
        head = v.astype(jnp.bfloat16).astype(jnp.float32)
        tail = v - head
        return (jax.lax.dot_general(head, rsel, (((1,), (0,)), ((), ())),
                                    preferred_element_type=jnp.float32)
                + jax.lax.dot_general(tail, rsel, (((1,), (0,)), ((), ())),
                                      preferred_element_type=jnp.float32))

    pos_exp = jax.lax.dot_general(pos_sel, rsel, (((1,), (0,)), ((), ())),
                                  preferred_element_type=jnp.float32)
    wgt_exp = expand(wgt)
    cmod = cmod_ref[0:1, :]
    disp = (pos_exp == cmod).astype(jnp.float32)
    disp_ref[0] = disp
    comb_ref[0] = disp * wgt_exp

    accm_ref[...] = accm_ref[...] + jnp.broadcast_to(colsum, accm_ref.shape)
    accp_ref[...] = accp_ref[...] + jnp.broadcast_to(
        jnp.sum(p, axis=0, keepdims=True), accp_ref.shape)

    @pl.when(i == NBLK - 1)
    def _():
        term = jnp.sum(accm_ref[0:1, :] * accp_ref[0:1, :])
        prev = jnp.where(b == 0, 0.0, aux_ref[0, 0])
        aux_ref[...] = jnp.broadcast_to(
            prev + term * (E / (B * T * T)), aux_ref.shape)


def kernel(x, w_gating):
    lanes = jnp.arange(EC, dtype=jnp.int32)
    rsel = (lanes // C == jnp.arange(E, dtype=jnp.int32)[:, None]
            ).astype(jnp.float32)
    cmod = jnp.broadcast_to((lanes % C).astype(jnp.float32), (8, EC))
    disp, comb, aux = pl.pallas_call(
        _router_kernel,
        grid=(B, NBLK),
        in_specs=[pl.BlockSpec((1, TBLK, DIM), lambda b, i: (b, i, 0)),
                  pl.BlockSpec((DIM, E), lambda b, i: (0, 0)),
                  pl.BlockSpec((E, EC), lambda b, i: (0, 0)),
                  pl.BlockSpec((8, EC), lambda b, i: (0, 0))],
        out_specs=[pl.BlockSpec((1, TBLK, EC), lambda b, i: (b, i, 0)),
                   pl.BlockSpec((1, TBLK, EC), lambda b, i: (b, i, 0)),
                   pl.BlockSpec((8, 128), lambda b, i: (0, 0))],
        out_shape=[jax.ShapeDtypeStruct((B, T, EC), jnp.float32),
                   jax.ShapeDtypeStruct((B, T, EC), jnp.float32),
                   jax.ShapeDtypeStruct((8, 128), jnp.float32)],
        scratch_shapes=[pltpu.VMEM((8, E), jnp.float32),
                        pltpu.VMEM((8, E), jnp.float32),
                        pltpu.VMEM((8, E), jnp.float32)],
        compiler_params=pltpu.CompilerParams(
            dimension_semantics=("arbitrary", "arbitrary")),
    )(x, w_gating, rsel, cmod)
    return (disp.reshape(B, T, E, C), comb.reshape(B, T, E, C),
            aux[0, 0].reshape(()))


# speedup vs baseline: 1.4314x; 1.1915x over previous
import math

import jax
import jax.numpy as jnp
from jax.experimental import pallas as pl
from jax.experimental.pallas import tpu as pltpu

B, T, DIM, E = 2, 2048, 1024, 64
THRESHOLD = 0.8
C = max(min(T, math.ceil(T * 1.25 / E)), 4)
TBLK = 512
NBLK = T // TBLK


EC = E * C


def _router_kernel(x_ref, w_ref, rsel_ref, cmod_ref, disp_ref, comb_ref,
                   aux_ref, carry_ref, accm_ref, accp_ref):
    b = pl.program_id(0)
    i = pl.program_id(1)

    @pl.when(i == 0)
    def _():
        carry_ref[...] = jnp.zeros_like(carry_ref)
        accm_ref[...] = jnp.zeros_like(accm_ref)
        accp_ref[...] = jnp.zeros_like(accp_ref)

    xb = x_ref[0]
    w = w_ref[...]
    gT = jax.lax.dot_general(w, xb, (((0,), (1,)), ((), ())),
                             preferred_element_type=jnp.float32)
    g = gT.T

    m = jnp.max(g, axis=-1, keepdims=True)
    ex = jnp.exp(g - m)
    tacc = ex
    width = E // 2
    while width >= 1:
        tacc = tacc[:, :width] + tacc[:, width:2 * width]
        width //= 2
    p = ex / tacc

    li = jax.lax.broadcasted_iota(jnp.int32, (1, E), 1)

    def lane_roll(a, sh):
        return pltpu.roll(a, sh % E, 1)

    a = p
    for k in (2, 4, 8, 16, 32, 64):
        j = k // 2
        while j >= 1:
            bitj0 = (li & j) == 0
            descd = (li & k) == 0
            part = jnp.where(bitj0, lane_roll(a, -j), lane_roll(a, j))
            mn = jnp.minimum(a, part)
            mx = jnp.maximum(a, part)
            a = jnp.where(bitj0 == descd, mx, mn)
            j //= 2
    psort = a

    si = jax.lax.broadcasted_iota(jnp.int32, (E, E), 0)
    sj = jax.lax.broadcasted_iota(jnp.int32, (E, E), 1)
    utri = (si <= sj).astype(jnp.float32)
    ustrict = (si < sj).astype(jnp.float32)

    def lane_scan(v, mat):
        head = v.astype(jnp.bfloat16).astype(jnp.float32)
        tail = v - head
        return (jax.lax.dot_general(head, mat, (((1,), (0,)), ((), ())),
                                    preferred_element_type=jnp.float32)
                + jax.lax.dot_general(tail, mat, (((1,), (0,)), ((), ())),
                                      preferred_element_type=jnp.float32))

    csum = lane_scan(psort, utri)
    csum = jnp.where(li == 1, lane_roll(psort, 1) + psort, csum)
    csum = jnp.where(li == 2, lane_roll(csum, 1) + psort, csum)

    kc = jnp.maximum(
        jnp.sum((csum < THRESHOLD).astype(jnp.float32), axis=-1,
                keepdims=True), 1.0)
    kcl = jnp.broadcast_to(kc, (TBLK, E))
    tau = jnp.min(jnp.where(li.astype(jnp.float32) < kcl, psort, jnp.inf),
                  axis=-1, keepdims=True)
    gt = (p > tau).astype(jnp.float32)
    eq = (p == tau).astype(jnp.float32)
    need = kc - jnp.sum(gt, axis=-1, keepdims=True)
    pre = jax.lax.dot_general(eq, ustrict, (((1,), (0,)), ((), ())),
                              preferred_element_type=jnp.float32)
    keepf = gt + eq * (pre < need).astype(jnp.float32)
    renorm = jnp.maximum(jnp.sum(keepf * p, axis=-1, keepdims=True), 1e-9)
    wgt = (keepf * p) / renorm

    ii = jax.lax.broadcasted_iota(jnp.int32, (TBLK, TBLK), 0)
    jj = jax.lax.broadcasted_iota(jnp.int32, (TBLK, TBLK), 1)
    lstrict = (ii > jj).astype(jnp.float32)
    pos_local = jax.lax.dot_general(lstrict, keepf, (((1,), (0,)), ((), ())),
                                    preferred_element_type=jnp.float32)
    carry = carry_ref[0:1, :]
    pos = pos_local + carry
    colsum = jnp.sum(keepf, axis=0, keepdims=True)
    carry_ref[...] = jnp.broadcast_to(carry + colsum, carry_ref.shape)

    keep_cap = keepf * (pos < float(C)).astype(jnp.float32)
    pos_sel = pos * keep_cap + float(C) * (1.0 - keep_cap)

    rsel = rsel_ref[...]

    pos_exp = jax.lax.dot_general(pos_sel, rsel, (((1,), (0,)), ((), ())),
                                  preferred_element_type=jnp.float32)
    wgt_exp = jax.lax.dot_general(wgt, rsel, (((1,), (0,)), ((), ())),
                                  preferred_element_type=jnp.float32)
    cmod = cmod_ref[0:1, :]
    disp = (pos_exp == cmod).astype(jnp.float32)
    disp_ref[0] = disp
    comb_ref[0] = disp * wgt_exp

    accm_ref[...] = accm_ref[...] + jnp.broadcast_to(colsum, accm_ref.shape)
    accp_ref[...] = accp_ref[...] + jnp.broadcast_to(
        jnp.sum(p, axis=0, keepdims=True), accp_ref.shape)

    @pl.when(i == NBLK - 1)
    def _():
        term = jnp.sum(accm_ref[0:1, :] * accp_ref[0:1, :])
        prev = jnp.where(b == 0, 0.0, aux_ref[0, 0])
        aux_ref[...] = jnp.broadcast_to(
            prev + term * (E / (B * T * T)), aux_ref.shape)


def kernel(x, w_gating):
    lanes = jnp.arange(EC, dtype=jnp.int32)
    rsel = (lanes // C == jnp.arange(E, dtype=jnp.int32)[:, None]
            ).astype(jnp.float32)
    cmod = jnp.broadcast_to((lanes % C).astype(jnp.float32), (8, EC))
    disp, comb, aux = pl.pallas_call(
        _router_kernel,
        grid=(B, NBLK),
        in_specs=[pl.BlockSpec((1, TBLK, DIM), lambda b, i: (b, i, 0)),
                  pl.BlockSpec((DIM, E), lambda b, i: (0, 0)),
                  pl.BlockSpec((E, EC), lambda b, i: (0, 0)),
                  pl.BlockSpec((8, EC), lambda b, i: (0, 0))],
        out_specs=[pl.BlockSpec((1, TBLK, EC), lambda b, i: (b, i, 0)),
                   pl.BlockSpec((1, TBLK, EC), lambda b, i: (b, i, 0)),
                   pl.BlockSpec((8, 128), lambda b, i: (0, 0))],
        out_shape=[jax.ShapeDtypeStruct((B, T, EC), jnp.float32),
                   jax.ShapeDtypeStruct((B, T, EC), jnp.float32),
                   jax.ShapeDtypeStruct((8, 128), jnp.float32)],
        scratch_shapes=[pltpu.VMEM((8, E), jnp.float32),
                        pltpu.VMEM((8, E), jnp.float32),
                        pltpu.VMEM((8, E), jnp.float32)],
        compiler_params=pltpu.CompilerParams(
            dimension_semantics=("arbitrary", "arbitrary")),
    )(x, w_gating, rsel, cmod)
    return (disp.reshape(B, T, E, C), comb.reshape(B, T, E, C),
            aux[0, 0].reshape(()))
